# baseline (speedup 1.0000x reference)
import functools

import jax
import jax.numpy as jnp
from jax import lax
from jax.experimental import pallas as pl
from jax.experimental.pallas import tpu as pltpu
from jax.experimental.pallas import tpu_sc as plsc

_BATCH = 16384
_DIM = 64
_PAIR = 2 * _DIM
_NC = 2
_NS = 16
_NW = _NC * _NS
_BPW = _BATCH // _NW
_GRP = 16
_IGRP = 8
_CHUNK = 64
_NCHUNK = _BPW // _CHUNK


def _softplus(x):
    v = jnp.exp(-jnp.abs(x))
    w = v / (v + 2.0)
    w2 = w * w
    log1p = 2.0 * w * (1.0 + w2 * (1.0 / 3.0 + w2 * (0.2 + w2 * (1.0 / 7.0 + w2 / 9.0))))
    return jnp.maximum(x, 0.0) + log1p


def _sc_body(h_idx, r_idx, t_idx, ent2, rel2, out_hbm,
             idx_h, idx_r, idx_t,
             h0, r0, t0, h1, r1, t1,
             scores, sem0, sem1):
    wid = lax.axis_index("s") * _NC + lax.axis_index("c")
    base = wid * _BPW

    pltpu.sync_copy(h_idx.at[pl.ds(base, _BPW)], idx_h.at[pl.ds(0, _BPW)])
    pltpu.sync_copy(r_idx.at[pl.ds(base, _BPW)], idx_r.at[pl.ds(0, _BPW)])
    pltpu.sync_copy(t_idx.at[pl.ds(base, _BPW)], idx_t.at[pl.ds(0, _BPW)])

    bufs = ((h0, r0, t0), (h1, r1, t1))
    sems = (sem0, sem1)
    lane = lax.iota(jnp.int32, _GRP)
    lane_masks = [lane == k for k in range(_GRP)]
    zeros = jnp.zeros((_GRP,), jnp.float32)

    def issue(c, slot):
        hb, rb, tb = bufs[slot]
        sem = sems[slot]

        def issue_body(g, carry):
            sl = pl.ds(g * _IGRP, _GRP)
            hv = idx_h[sl] >> 1
            rv = idx_r[sl] >> 1
            tv = idx_t[sl] >> 1
            jb = g * _IGRP - c * _CHUNK
            for k in range(_IGRP):
                j = jb + k
                pltpu.async_copy(ent2.at[hv[k]], hb.at[j], sem)
                pltpu.async_copy(rel2.at[rv[k]], rb.at[j], sem)
                pltpu.async_copy(ent2.at[tv[k]], tb.at[j], sem)
            return carry

        cg = _CHUNK // _IGRP
        lax.fori_loop(c * cg, (c + 1) * cg, issue_body, 0)

    def drain(slot):
        hb, rb, tb = bufs[slot]
        sem = sems[slot]
        src = ent2.at[pl.ds(0, _CHUNK)]
        pltpu.make_async_copy(src, hb, sem).wait()
        pltpu.make_async_copy(src, rb, sem).wait()
        pltpu.make_async_copy(src, tb, sem).wait()

    def compute(c, slot):
        hb, rb, tb = bufs[slot]

        def dot_body(g, carry):
            sl = pl.ds(g * _GRP, _GRP)
            hp = (idx_h[sl] & 1) * _DIM
            rp = (idx_r[sl] & 1) * _DIM
            tp = (idx_t[sl] & 1) * _DIM
            jb = g * _GRP - c * _CHUNK
            out = zeros
            for k in range(_GRP):
                j = jb + k
                ho, ro, to = hp[k], rp[k], tp[k]
                acc = (hb[j, pl.ds(ho, 16)] * rb[j, pl.ds(ro, 16)]
                       * tb[j, pl.ds(to, 16)])
                for d in range(1, _DIM // 16):
                    acc = acc + (hb[j, pl.ds(ho + d * 16, 16)]
                                 * rb[j, pl.ds(ro + d * 16, 16)]
                                 * tb[j, pl.ds(to + d * 16, 16)])
                s = jnp.sum(acc)
                out = jnp.where(lane_masks[k], s, out)
            scores[pl.ds(g * _GRP, _GRP)] = _softplus(out)
            return carry

        cg = _CHUNK // _GRP
        lax.fori_loop(c * cg, (c + 1) * cg, dot_body, 0)

    issue(0, 0)
    for c in range(_NCHUNK):
        if c + 1 < _NCHUNK:
            issue(c + 1, (c + 1) & 1)
        drain(c & 1)
        compute(c, c & 1)

    pltpu.sync_copy(scores, out_hbm.at[pl.ds(base, _BPW)])


@jax.jit
def _focus_e_sc(h_idx, r_idx, t_idx, ent2, rel2):
    mesh = plsc.VectorSubcoreMesh(core_axis_name="c", subcore_axis_name="s")
    rows = pltpu.VMEM((_CHUNK, _PAIR), jnp.float32)
    kern = functools.partial(
        pl.kernel,
        mesh=mesh,
        compiler_params=pltpu.CompilerParams(
            needs_layout_passes=False, use_tc_tiling_on_sc=True),
        out_type=jax.ShapeDtypeStruct((_BATCH,), jnp.float32),
        scratch_types=[
            pltpu.VMEM((_BPW + _GRP,), jnp.int32),
            pltpu.VMEM((_BPW + _GRP,), jnp.int32),
            pltpu.VMEM((_BPW + _GRP,), jnp.int32),
            rows, rows, rows, rows, rows, rows,
            pltpu.VMEM((_BPW,), jnp.float32),
            pltpu.SemaphoreType.DMA,
            pltpu.SemaphoreType.DMA,
        ],
    )(_sc_body)
    return kern(h_idx, r_idx, t_idx, ent2, rel2)


def kernel(triples, ent_emb, rel_emb):
    idx = triples.astype(jnp.int32)
    ent2 = ent_emb.reshape(ent_emb.shape[0] // 2, _PAIR)
    rel2 = rel_emb.reshape(rel_emb.shape[0] // 2, _PAIR)
    return _focus_e_sc(idx[:, 0], idx[:, 1], idx[:, 2], ent2, rel2)


# speedup vs baseline: 1.6060x; 1.6060x over previous
import functools

import jax
import jax.numpy as jnp
from jax import lax
from jax.experimental import pallas as pl
from jax.experimental.pallas import tpu as pltpu
from jax.experimental.pallas import tpu_sc as plsc

_BATCH = 16384
_DIM = 64
_PAIR = 2 * _DIM
_NC = 2
_NS = 16
_NW = _NC * _NS
_BPW = _BATCH // _NW
_GRP = 16
_IGRP = 8
_CHUNK = 64
_NCHUNK = _BPW // _CHUNK

_TP = 2048
_NENT = 1000000
_TGRID = (_NENT + 2 * _TP - 1) // (2 * _TP)
_EROWS = _TGRID * _TP


def _softplus(x):
    v = jnp.exp(-jnp.abs(x))
    w = v / (v + 2.0)
    w2 = w * w
    log1p = 2.0 * w * (1.0 + w2 * (1.0 / 3.0 + w2 * (0.2 + w2 * (1.0 / 7.0 + w2 / 9.0))))
    return jnp.maximum(x, 0.0) + log1p


def _tct_body(x_ref, y_ref):
    a = x_ref[...].T
    y_ref[...] = jnp.concatenate([a[:_TP], a[_TP:]], axis=1)


def _transpose_pairs(tT):
    return pl.pallas_call(
        _tct_body,
        grid=(_TGRID,),
        in_specs=[pl.BlockSpec((_DIM, 2 * _TP), lambda i: (0, i))],
        out_specs=pl.BlockSpec((_TP, _PAIR), lambda i: (i, 0)),
        out_shape=jax.ShapeDtypeStruct((_EROWS, _PAIR), jnp.float32),
    )(tT)


def _sc_body(h_idx, r_idx, t_idx, ent2, rel_emb, out_hbm,
             idx_h, idx_r, idx_t,
             h0, r0, t0, h1, r1, t1,
             scores, sem0, sem1):
    wid = lax.axis_index("s") * _NC + lax.axis_index("c")
    base = wid * _BPW

    pltpu.sync_copy(h_idx.at[pl.ds(base, _BPW)], idx_h.at[pl.ds(0, _BPW)])
    pltpu.sync_copy(r_idx.at[pl.ds(base, _BPW)], idx_r.at[pl.ds(0, _BPW)])
    pltpu.sync_copy(t_idx.at[pl.ds(base, _BPW)], idx_t.at[pl.ds(0, _BPW)])

    bufs = ((h0, r0, t0), (h1, r1, t1))
    sems = (sem0, sem1)
    lane = lax.iota(jnp.int32, _GRP)
    lane_masks = [lane == k for k in range(_GRP)]
    zeros = jnp.zeros((_GRP,), jnp.float32)

    def pair_id(v):
        return ((v >> 12) << 11) + (v & (_TP - 1))

    def issue(c, slot):
        hb, rb, tb = bufs[slot]
        sem = sems[slot]

        def issue_body(g, carry):
            sl = pl.ds(g * _IGRP, _GRP)
            hv = pair_id(idx_h[sl])
            rv = idx_r[sl]
            tv = pair_id(idx_t[sl])
            jb = g * _IGRP - c * _CHUNK
            for k in range(_IGRP):
                j = jb + k
                pltpu.async_copy(ent2.at[hv[k]], hb.at[j], sem)
                pltpu.async_copy(rel_emb.at[rv[k]], rb.at[j], sem)
                pltpu.async_copy(ent2.at[tv[k]], tb.at[j], sem)
            return carry

        cg = _CHUNK // _IGRP
        lax.fori_loop(c * cg, (c + 1) * cg, issue_body, 0)

    def drain(slot):
        hb, rb, tb = bufs[slot]
        sem = sems[slot]
        srcp = ent2.at[pl.ds(0, _CHUNK)]
        pltpu.make_async_copy(srcp, hb, sem).wait()
        pltpu.make_async_copy(rel_emb.at[pl.ds(0, _CHUNK)], rb, sem).wait()
        pltpu.make_async_copy(srcp, tb, sem).wait()

    def compute(c, slot):
        hb, rb, tb = bufs[slot]

        def dot_body(g, carry):
            sl = pl.ds(g * _GRP, _GRP)
            hp = ((idx_h[sl] >> 11) & 1) * _DIM
            tp = ((idx_t[sl] >> 11) & 1) * _DIM
            jb = g * _GRP - c * _CHUNK
            out = zeros
            for k in range(_GRP):
                j = jb + k
                ho, to = hp[k], tp[k]
                acc = (hb[j, pl.ds(ho, 16)] * rb[j, pl.ds(0, 16)]
                       * tb[j, pl.ds(to, 16)])
                for d in range(1, _DIM // 16):
                    acc = acc + (hb[j, pl.ds(ho + d * 16, 16)]
                                 * rb[j, pl.ds(d * 16, 16)]
                                 * tb[j, pl.ds(to + d * 16, 16)])
                s = jnp.sum(acc)
                out = jnp.where(lane_masks[k], s, out)
            scores[pl.ds(g * _GRP, _GRP)] = _softplus(out)
            return carry

        cg = _CHUNK // _GRP
        lax.fori_loop(c * cg, (c + 1) * cg, dot_body, 0)

    issue(0, 0)
    for c in range(_NCHUNK):
        if c + 1 < _NCHUNK:
            issue(c + 1, (c + 1) & 1)
        drain(c & 1)
        compute(c, c & 1)

    pltpu.sync_copy(scores, out_hbm.at[pl.ds(base, _BPW)])


@jax.jit
def _focus_e(h_idx, r_idx, t_idx, entT, rel_emb):
    ent2 = _transpose_pairs(entT)
    mesh = plsc.VectorSubcoreMesh(core_axis_name="c", subcore_axis_name="s")
    prow = pltpu.VMEM((_CHUNK, _PAIR), jnp.float32)
    rrow = pltpu.VMEM((_CHUNK, _DIM), jnp.float32)
    kern = functools.partial(
        pl.kernel,
        mesh=mesh,
        compiler_params=pltpu.CompilerParams(
            needs_layout_passes=False, use_tc_tiling_on_sc=True),
        out_type=jax.ShapeDtypeStruct((_BATCH,), jnp.float32),
        scratch_types=[
            pltpu.VMEM((_BPW + _GRP,), jnp.int32),
            pltpu.VMEM((_BPW + _GRP,), jnp.int32),
            pltpu.VMEM((_BPW + _GRP,), jnp.int32),
            prow, rrow, prow, prow, rrow, prow,
            pltpu.VMEM((_BPW,), jnp.float32),
            pltpu.SemaphoreType.DMA,
            pltpu.SemaphoreType.DMA,
        ],
    )(_sc_body)
    return kern(h_idx, r_idx, t_idx, ent2, rel_emb)


def kernel(triples, ent_emb, rel_emb):
    idx = triples.astype(jnp.int32)
    return _focus_e(idx[:, 0], idx[:, 1], idx[:, 2], ent_emb.T, rel_emb)


# speedup vs baseline: 2.5450x; 1.5847x over previous
import functools

import jax
import jax.numpy as jnp
from jax import lax
from jax.experimental import pallas as pl
from jax.experimental.pallas import tpu as pltpu
from jax.experimental.pallas import tpu_sc as plsc

_BATCH = 16384
_DIM = 64
_PAIR = 2 * _DIM
_NC = 2
_NS = 16
_NW = _NC * _NS
_BPW = _BATCH // _NW
_GRP = 16
_IGRP = 8
_CHUNK = 64
_NCHUNK = _BPW // _CHUNK

_TP = 2048
_NENT = 1000000
_TGRID = (_NENT + 2 * _TP - 1) // (2 * _TP)
_EROWS = _TGRID * _TP
_RHALF = 500000


def _softplus(x):
    v = jnp.exp(-jnp.abs(x))
    w = v / (v + 2.0)
    w2 = w * w
    log1p = 2.0 * w * (1.0 + w2 * (1.0 / 3.0 + w2 * (0.2 + w2 * (1.0 / 7.0 + w2 / 9.0))))
    return jnp.maximum(x, 0.0) + log1p


def _tct_body(x_ref, y_ref):
    a = x_ref[...].T
    y_ref[...] = jnp.concatenate([a[:_TP], a[_TP:]], axis=1)


def _transpose_pairs(tT):
    return pl.pallas_call(
        _tct_body,
        grid=(_TGRID,),
        in_specs=[pl.BlockSpec((_DIM, 2 * _TP), lambda i: (0, i))],
        out_specs=pl.BlockSpec((_TP, _PAIR), lambda i: (i, 0)),
        out_shape=jax.ShapeDtypeStruct((_EROWS, _PAIR), jnp.float32),
    )(tT)


def _sc_body(h_idx, r_idx, t_idx, ent2, rel_emb, out_hbm,
             idx_h, idx_r, idx_t,
             h0, r0, t0, h1, r1, t1,
             scores, sem0, sem1):
    wid = lax.axis_index("s") * _NC + lax.axis_index("c")
    base = wid * _BPW

    pltpu.sync_copy(h_idx.at[pl.ds(base, _BPW)], idx_h.at[pl.ds(0, _BPW)])
    pltpu.sync_copy(r_idx.at[pl.ds(base, _BPW)], idx_r.at[pl.ds(0, _BPW)])
    pltpu.sync_copy(t_idx.at[pl.ds(base, _BPW)], idx_t.at[pl.ds(0, _BPW)])

    bufs = ((h0, r0, t0), (h1, r1, t1))
    sems = (sem0, sem1)
    lane = lax.iota(jnp.int32, _GRP)
    lane_masks = [lane == k for k in range(_GRP)]
    zeros = jnp.zeros((_GRP,), jnp.float32)

    def pair_id(v):
        return ((v >> 12) << 11) + (v & (_TP - 1))

    def issue(c, slot):
        hb, rb, tb = bufs[slot]
        sem = sems[slot]

        def issue_body(g, carry):
            sl = pl.ds(g * _IGRP, _GRP)
            hv = pair_id(idx_h[sl])
            rv = idx_r[sl]
            ra = (rv >= _RHALF).astype(jnp.int32)
            rr = rv - ra * _RHALF
            tv = pair_id(idx_t[sl])
            jb = g * _IGRP - c * _CHUNK
            for k in range(_IGRP):
                j = jb + k
                pltpu.async_copy(ent2.at[hv[k]], hb.at[j], sem)
                pltpu.async_copy(rel_emb.at[ra[k], rr[k]], rb.at[j], sem)
                pltpu.async_copy(ent2.at[tv[k]], tb.at[j], sem)
            return carry

        cg = _CHUNK // _IGRP
        lax.fori_loop(c * cg, (c + 1) * cg, issue_body, 0)

    def drain(slot):
        hb, rb, tb = bufs[slot]
        sem = sems[slot]
        srcp = ent2.at[pl.ds(0, _CHUNK)]
        pltpu.make_async_copy(srcp, hb, sem).wait()
        pltpu.make_async_copy(rel_emb.at[0, pl.ds(0, _CHUNK)], rb, sem).wait()
        pltpu.make_async_copy(srcp, tb, sem).wait()

    def compute(c, slot):
        hb, rb, tb = bufs[slot]

        def dot_body(g, carry):
            sl = pl.ds(g * _GRP, _GRP)
            hp = ((idx_h[sl] >> 11) & 1) * _DIM
            tp = ((idx_t[sl] >> 11) & 1) * _DIM
            jb = g * _GRP - c * _CHUNK
            out = zeros
            for k in range(_GRP):
                j = jb + k
                ho, to = hp[k], tp[k]
                acc = (hb[j, pl.ds(ho, 16)] * rb[j, pl.ds(0, 16)]
                       * tb[j, pl.ds(to, 16)])
                for d in range(1, _DIM // 16):
                    acc = acc + (hb[j, pl.ds(ho + d * 16, 16)]
                                 * rb[j, pl.ds(d * 16, 16)]
                                 * tb[j, pl.ds(to + d * 16, 16)])
                s = jnp.sum(acc)
                out = jnp.where(lane_masks[k], s, out)
            scores[pl.ds(g * _GRP, _GRP)] = _softplus(out)
            return carry

        cg = _CHUNK // _GRP
        lax.fori_loop(c * cg, (c + 1) * cg, dot_body, 0)

    issue(0, 0)
    for c in range(_NCHUNK):
        if c + 1 < _NCHUNK:
            issue(c + 1, (c + 1) & 1)
        drain(c & 1)
        compute(c, c & 1)

    pltpu.sync_copy(scores, out_hbm.at[pl.ds(base, _BPW)])


@jax.jit
def _focus_e(h_idx, r_idx, t_idx, entT, rel_emb):
    ent2 = _transpose_pairs(entT)
    mesh = plsc.VectorSubcoreMesh(core_axis_name="c", subcore_axis_name="s")
    prow = pltpu.VMEM((_CHUNK, _PAIR), jnp.float32)
    rrow = pltpu.VMEM((_CHUNK, _DIM), jnp.float32)
    kern = functools.partial(
        pl.kernel,
        mesh=mesh,
        compiler_params=pltpu.CompilerParams(
            needs_layout_passes=False, use_tc_tiling_on_sc=True),
        out_type=jax.ShapeDtypeStruct((_BATCH,), jnp.float32),
        scratch_types=[
            pltpu.VMEM((_BPW + _GRP,), jnp.int32),
            pltpu.VMEM((_BPW + _GRP,), jnp.int32),
            pltpu.VMEM((_BPW + _GRP,), jnp.int32),
            prow, rrow, prow, prow, rrow, prow,
            pltpu.VMEM((_BPW,), jnp.float32),
            pltpu.SemaphoreType.DMA,
            pltpu.SemaphoreType.DMA,
        ],
    )(_sc_body)
    return kern(h_idx, r_idx, t_idx, ent2, rel_emb)


def kernel(triples, ent_emb, rel_emb):
    idx = triples.astype(jnp.int32)
    return _focus_e(idx[:, 0], idx[:, 1], idx[:, 2], ent_emb.T,
                    rel_emb.reshape(2, _RHALF, _DIM))


# speedup vs baseline: 2.9602x; 1.1631x over previous
import functools

import jax
import jax.numpy as jnp
from jax import lax
from jax.experimental import pallas as pl
from jax.experimental.pallas import tpu as pltpu
from jax.experimental.pallas import tpu_sc as plsc

_BATCH = 16384
_DIM = 64
_PAIR = 2 * _DIM
_NC = 2
_NS = 16
_NW = _NC * _NS
_BPW = _BATCH // _NW
_GRP = 16
_IGRP = 8
_CHUNK = 64
_NCHUNK = _BPW // _CHUNK

_TP = 4096
_NENT = 1000000
_TGRID = (_NENT + 2 * _TP - 1) // (2 * _TP)
_EROWS = _TGRID * _TP
_RHALF = 500000


def _softplus(x):
    v = jnp.exp(-jnp.abs(x))
    w = v / (v + 2.0)
    w2 = w * w
    log1p = 2.0 * w * (1.0 + w2 * (1.0 / 3.0 + w2 * (0.2 + w2 * (1.0 / 7.0 + w2 / 9.0))))
    return jnp.maximum(x, 0.0) + log1p


def _tct_body(x_ref, y_ref):
    a = x_ref[...].T
    y_ref[...] = jnp.concatenate([a[:_TP], a[_TP:]], axis=1)


def _transpose_pairs(tT):
    return pl.pallas_call(
        _tct_body,
        grid=(_TGRID,),
        in_specs=[pl.BlockSpec((_DIM, 2 * _TP), lambda i: (0, i))],
        out_specs=pl.BlockSpec((_TP, _PAIR), lambda i: (i, 0)),
        out_shape=jax.ShapeDtypeStruct((_EROWS, _PAIR), jnp.float32),
    )(tT)


def _sc_body(h_idx, r_idx, t_idx, ent2, rel_emb, out_hbm,
             idx_h, idx_r, idx_t,
             h0, r0, t0, h1, r1, t1,
             scores, sem0, sem1):
    wid = lax.axis_index("s") * _NC + lax.axis_index("c")
    base = wid * _BPW

    pltpu.sync_copy(h_idx.at[pl.ds(base, _BPW)], idx_h.at[pl.ds(0, _BPW)])
    pltpu.sync_copy(r_idx.at[pl.ds(base, _BPW)], idx_r.at[pl.ds(0, _BPW)])
    pltpu.sync_copy(t_idx.at[pl.ds(base, _BPW)], idx_t.at[pl.ds(0, _BPW)])

    bufs = ((h0, r0, t0), (h1, r1, t1))
    sems = (sem0, sem1)
    lane = lax.iota(jnp.int32, _GRP)
    lane_masks = [lane == k for k in range(_GRP)]
    zeros = jnp.zeros((_GRP,), jnp.float32)

    def pair_id(v):
        return ((v >> 13) << 12) + (v & (_TP - 1))

    def issue(c, slot):
        hb, rb, tb = bufs[slot]
        sem = sems[slot]

        def issue_body(g, carry):
            sl = pl.ds(g * _IGRP, _GRP)
            hv = pair_id(idx_h[sl])
            rv = idx_r[sl]
            ra = (rv >= _RHALF).astype(jnp.int32)
            rr = rv - ra * _RHALF
            tv = pair_id(idx_t[sl])
            jb = g * _IGRP - c * _CHUNK
            for k in range(_IGRP):
                j = jb + k
                pltpu.async_copy(ent2.at[hv[k]], hb.at[j], sem)
                pltpu.async_copy(rel_emb.at[ra[k], rr[k]], rb.at[j], sem)
                pltpu.async_copy(ent2.at[tv[k]], tb.at[j], sem)
            return carry

        cg = _CHUNK // _IGRP
        lax.fori_loop(c * cg, (c + 1) * cg, issue_body, 0)

    def drain(slot):
        hb, rb, tb = bufs[slot]
        sem = sems[slot]
        srcp = ent2.at[pl.ds(0, _CHUNK)]
        pltpu.make_async_copy(srcp, hb, sem).wait()
        pltpu.make_async_copy(rel_emb.at[0, pl.ds(0, _CHUNK)], rb, sem).wait()
        pltpu.make_async_copy(srcp, tb, sem).wait()

    def compute(c, slot):
        hb, rb, tb = bufs[slot]

        def dot_body(g, carry):
            sl = pl.ds(g * _GRP, _GRP)
            hp = ((idx_h[sl] >> 12) & 1) * _DIM
            tp = ((idx_t[sl] >> 12) & 1) * _DIM
            jb = g * _GRP - c * _CHUNK
            out = zeros
            for k in range(_GRP):
                j = jb + k
                ho, to = hp[k], tp[k]
                acc = (hb[j, pl.ds(ho, 16)] * rb[j, pl.ds(0, 16)]
                       * tb[j, pl.ds(to, 16)])
                for d in range(1, _DIM // 16):
                    acc = acc + (hb[j, pl.ds(ho + d * 16, 16)]
                                 * rb[j, pl.ds(d * 16, 16)]
                                 * tb[j, pl.ds(to + d * 16, 16)])
                s = jnp.sum(acc)
                out = jnp.where(lane_masks[k], s, out)
            scores[pl.ds(g * _GRP, _GRP)] = _softplus(out)
            return carry

        cg = _CHUNK // _GRP
        lax.fori_loop(c * cg, (c + 1) * cg, dot_body, 0)

    issue(0, 0)
    for c in range(_NCHUNK):
        if c + 1 < _NCHUNK:
            issue(c + 1, (c + 1) & 1)
        drain(c & 1)
        compute(c, c & 1)

    pltpu.sync_copy(scores, out_hbm.at[pl.ds(base, _BPW)])


@jax.jit
def _focus_e(h_idx, r_idx, t_idx, entT, rel_emb):
    ent2 = _transpose_pairs(entT)
    mesh = plsc.VectorSubcoreMesh(core_axis_name="c", subcore_axis_name="s")
    prow = pltpu.VMEM((_CHUNK, _PAIR), jnp.float32)
    rrow = pltpu.VMEM((_CHUNK, _DIM), jnp.float32)
    kern = functools.partial(
        pl.kernel,
        mesh=mesh,
        compiler_params=pltpu.CompilerParams(
            needs_layout_passes=False, use_tc_tiling_on_sc=True),
        out_type=jax.ShapeDtypeStruct((_BATCH,), jnp.float32),
        scratch_types=[
            pltpu.VMEM((_BPW + _GRP,), jnp.int32),
            pltpu.VMEM((_BPW + _GRP,), jnp.int32),
            pltpu.VMEM((_BPW + _GRP,), jnp.int32),
            prow, rrow, prow, prow, rrow, prow,
            pltpu.VMEM((_BPW,), jnp.float32),
            pltpu.SemaphoreType.DMA,
            pltpu.SemaphoreType.DMA,
        ],
    )(_sc_body)
    return kern(h_idx, r_idx, t_idx, ent2, rel_emb)


def kernel(triples, ent_emb, rel_emb):
    idx = triples.astype(jnp.int32)
    return _focus_e(idx[:, 0], idx[:, 1], idx[:, 2], ent_emb.T,
                    rel_emb.reshape(2, _RHALF, _DIM))


# speedup vs baseline: 2.9608x; 1.0002x over previous
import functools

import jax
import jax.numpy as jnp
from jax import lax
from jax.experimental import pallas as pl
from jax.experimental.pallas import tpu as pltpu
from jax.experimental.pallas import tpu_sc as plsc

_BATCH = 16384
_DIM = 64
_PAIR = 2 * _DIM
_NC = 2
_NS = 16
_NW = _NC * _NS
_BPW = _BATCH // _NW
_GRP = 16
_IGRP = 8
_CHUNK = 64
_NCHUNK = _BPW // _CHUNK

_TP = 8192
_NENT = 1000000
_TGRID = (_NENT + 2 * _TP - 1) // (2 * _TP)
_EROWS = _TGRID * _TP
_RHALF = 500000


def _softplus(x):
    v = jnp.exp(-jnp.abs(x))
    w = v / (v + 2.0)
    w2 = w * w
    log1p = 2.0 * w * (1.0 + w2 * (1.0 / 3.0 + w2 * (0.2 + w2 * (1.0 / 7.0 + w2 / 9.0))))
    return jnp.maximum(x, 0.0) + log1p


def _tct_body(x_ref, y_ref):
    a = x_ref[...].T
    y_ref[...] = jnp.concatenate([a[:_TP], a[_TP:]], axis=1)


def _transpose_pairs(tT):
    return pl.pallas_call(
        _tct_body,
        grid=(_TGRID,),
        in_specs=[pl.BlockSpec((_DIM, 2 * _TP), lambda i: (0, i))],
        out_specs=pl.BlockSpec((_TP, _PAIR), lambda i: (i, 0)),
        out_shape=jax.ShapeDtypeStruct((_EROWS, _PAIR), jnp.float32),
    )(tT)


def _sc_body(h_idx, r_idx, t_idx, ent2, rel_emb, out_hbm,
             idx_h, idx_r, idx_t,
             h0, r0, t0, h1, r1, t1,
             scores, sem0, sem1):
    wid = lax.axis_index("s") * _NC + lax.axis_index("c")
    base = wid * _BPW

    pltpu.sync_copy(h_idx.at[pl.ds(base, _BPW)], idx_h.at[pl.ds(0, _BPW)])
    pltpu.sync_copy(r_idx.at[pl.ds(base, _BPW)], idx_r.at[pl.ds(0, _BPW)])
    pltpu.sync_copy(t_idx.at[pl.ds(base, _BPW)], idx_t.at[pl.ds(0, _BPW)])

    bufs = ((h0, r0, t0), (h1, r1, t1))
    sems = (sem0, sem1)
    lane = lax.iota(jnp.int32, _GRP)
    lane_masks = [lane == k for k in range(_GRP)]
    zeros = jnp.zeros((_GRP,), jnp.float32)

    def pair_id(v):
        return ((v >> 14) << 13) + (v & (_TP - 1))

    def issue(c, slot):
        hb, rb, tb = bufs[slot]
        sem = sems[slot]

        def issue_body(g, carry):
            sl = pl.ds(g * _IGRP, _GRP)
            hv = pair_id(idx_h[sl])
            rv = idx_r[sl]
            ra = (rv >= _RHALF).astype(jnp.int32)
            rr = rv - ra * _RHALF
            tv = pair_id(idx_t[sl])
            jb = g * _IGRP - c * _CHUNK
            for k in range(_IGRP):
                j = jb + k
                pltpu.async_copy(ent2.at[hv[k]], hb.at[j], sem)
                pltpu.async_copy(rel_emb.at[ra[k], rr[k]], rb.at[j], sem)
                pltpu.async_copy(ent2.at[tv[k]], tb.at[j], sem)
            return carry

        cg = _CHUNK // _IGRP
        lax.fori_loop(c * cg, (c + 1) * cg, issue_body, 0)

    def drain(slot):
        hb, rb, tb = bufs[slot]
        sem = sems[slot]
        srcp = ent2.at[pl.ds(0, _CHUNK)]
        pltpu.make_async_copy(srcp, hb, sem).wait()
        pltpu.make_async_copy(rel_emb.at[0, pl.ds(0, _CHUNK)], rb, sem).wait()
        pltpu.make_async_copy(srcp, tb, sem).wait()

    def compute(c, slot):
        hb, rb, tb = bufs[slot]

        def dot_body(g, carry):
            sl = pl.ds(g * _GRP, _GRP)
            hp = ((idx_h[sl] >> 13) & 1) * _DIM
            tp = ((idx_t[sl] >> 13) & 1) * _DIM
            jb = g * _GRP - c * _CHUNK
            out = zeros
            for k in range(_GRP):
                j = jb + k
                ho, to = hp[k], tp[k]
                acc = (hb[j, pl.ds(ho, 16)] * rb[j, pl.ds(0, 16)]
                       * tb[j, pl.ds(to, 16)])
                for d in range(1, _DIM // 16):
                    acc = acc + (hb[j, pl.ds(ho + d * 16, 16)]
                                 * rb[j, pl.ds(d * 16, 16)]
                                 * tb[j, pl.ds(to + d * 16, 16)])
                s = jnp.sum(acc)
                out = jnp.where(lane_masks[k], s, out)
            scores[pl.ds(g * _GRP, _GRP)] = _softplus(out)
            return carry

        cg = _CHUNK // _GRP
        lax.fori_loop(c * cg, (c + 1) * cg, dot_body, 0)

    issue(0, 0)
    for c in range(_NCHUNK):
        if c + 1 < _NCHUNK:
            issue(c + 1, (c + 1) & 1)
        drain(c & 1)
        compute(c, c & 1)

    pltpu.sync_copy(scores, out_hbm.at[pl.ds(base, _BPW)])


@jax.jit
def _focus_e(h_idx, r_idx, t_idx, entT, rel_emb):
    ent2 = _transpose_pairs(entT)
    mesh = plsc.VectorSubcoreMesh(core_axis_name="c", subcore_axis_name="s")
    prow = pltpu.VMEM((_CHUNK, _PAIR), jnp.float32)
    rrow = pltpu.VMEM((_CHUNK, _DIM), jnp.float32)
    kern = functools.partial(
        pl.kernel,
        mesh=mesh,
        compiler_params=pltpu.CompilerParams(
            needs_layout_passes=False, use_tc_tiling_on_sc=True),
        out_type=jax.ShapeDtypeStruct((_BATCH,), jnp.float32),
        scratch_types=[
            pltpu.VMEM((_BPW + _GRP,), jnp.int32),
            pltpu.VMEM((_BPW + _GRP,), jnp.int32),
            pltpu.VMEM((_BPW + _GRP,), jnp.int32),
            prow, rrow, prow, prow, rrow, prow,
            pltpu.VMEM((_BPW,), jnp.float32),
            pltpu.SemaphoreType.DMA,
            pltpu.SemaphoreType.DMA,
        ],
    )(_sc_body)
    return kern(h_idx, r_idx, t_idx, ent2, rel_emb)


def kernel(triples, ent_emb, rel_emb):
    idx = triples.astype(jnp.int32)
    return _focus_e(idx[:, 0], idx[:, 1], idx[:, 2], ent_emb.T,
                    rel_emb.reshape(2, _RHALF, _DIM))
